# Initial kernel scaffold; baseline (speedup 1.0000x reference)
import jax
import jax.numpy as jnp
from jax.experimental import pallas as pl


def kernel(hidden, W, b):
    raise NotImplementedError("write your pallas kernel here")


# speedup vs baseline: 16.7123x; 16.7123x over previous
import jax
import jax.numpy as jnp
from jax.experimental import pallas as pl

B = 128
H = 256
V = 100000
S = 3
K = 50
P = 0.95
TP = 2048
NT = 50
VP = NT * TP
NC = TP // 128
NCH = NT * NC
NG = 56
NEG = -1e30
RB = 16
RB5 = 8


def _k1(lh_ref, w_ref, b_ref, l2_ref, l4_ref):
    l = jnp.dot(lh_ref[...], w_ref[...], preferred_element_type=jnp.float32)
    lp = l + b_ref[...]
    l2_ref[...] = lp
    for j in range(NC):
        l4_ref[:, j, :] = lp[:, j * 128:(j + 1) * 128]


def _k3(l4_ref, gp_ref):
    x3 = l4_ref[...]
    cm = jnp.max(x3, axis=2)
    iota = jax.lax.broadcasted_iota(jnp.int32, (RB, NCH), 1)
    jio = jax.lax.broadcasted_iota(jnp.int32, (RB, NG), 1)

    def step(j, carry):
        c, acc = carry
        m = jnp.max(c, axis=1, keepdims=True)
        cand = jnp.where(c == m, iota, NCH)
        idx = jnp.min(cand, axis=1, keepdims=True)
        c = jnp.where(iota == idx, -jnp.inf, c)
        acc = jnp.where(jio == j, idx, acc)
        return c, acc

    _, ci = jax.lax.fori_loop(
        0, NG, step, (cm, jnp.zeros((RB, NG), jnp.int32)))
    iota2 = jax.lax.broadcasted_iota(jnp.int32, (NG, NCH), 1)
    for r in range(RB):
        ci2 = ci[r].reshape(NG, 1)
        oh = (ci2 == iota2).astype(jnp.float32)
        gp_ref[r] = jnp.dot(oh, x3[r], preferred_element_type=jnp.float32,
                            precision=jax.lax.Precision.HIGHEST)


def _k4(gp_ref, m_ref, th_ref, d_ref):
    x = gp_ref[...]
    i1 = jax.lax.broadcasted_iota(jnp.int32, (B, NG, 128), 1)
    i2 = jax.lax.broadcasted_iota(jnp.int32, (B, NG, 128), 2)
    fid = i1 * 128 + i2
    jio = jax.lax.broadcasted_iota(jnp.int32, (B, NG), 1)

    def step(j, carry):
        x, acc = carry
        m = jnp.max(jnp.max(x, axis=2), axis=1, keepdims=True)
        cand = jnp.where(x == m[:, :, None], fid, NG * 128)
        cidx = jnp.min(jnp.min(cand, axis=2), axis=1, keepdims=True)
        x = jnp.where(fid == cidx[:, :, None], -jnp.inf, x)
        acc = jnp.where(jio == j, m, acc)
        return x, acc

    _, sv = jax.lax.fori_loop(
        0, NG, step, (x, jnp.zeros((B, NG), jnp.float32)))
    M = sv[:, 0:1]
    kv = sv[:, K - 1:K]
    e = jnp.where(sv >= kv, jnp.exp(sv - M), 0.0)
    D = jnp.sum(e, axis=1, keepdims=True)
    pv = e / D
    tri = (jax.lax.broadcasted_iota(jnp.int32, (NG, NG), 0)
           <= jax.lax.broadcasted_iota(jnp.int32, (NG, NG), 1)
           ).astype(jnp.float32)
    cum = jnp.dot(pv, tri, preferred_element_type=jnp.float32,
                  precision=jax.lax.Precision.HIGHEST)
    msk = (cum > P) & (jio > 0)
    th = jnp.min(jnp.where(msk, jnp.inf, sv), axis=1, keepdims=True)
    dfin = jnp.sum(jnp.where(sv >= th, e, 0.0), axis=1, keepdims=True)
    m_ref[0] = M
    th_ref[0] = th
    d_ref[0] = dfin


def _k5(l2_ref, u_ref, m_ref, th_ref, d_ref, pr_ref, bi_ref):
    l = l2_ref[...]
    M = m_ref[0]
    th = th_ref[0]
    D = d_ref[0]
    p = jnp.where(l >= th, jnp.exp(l - M) / D, 0.0)
    pr_ref[...] = p
    u = u_ref[...]
    g = -jnp.log(-jnp.log(u))
    sc = jnp.log(p + 1e-30) + g
    tb = jnp.max(sc, axis=1, keepdims=True)
    li = jax.lax.broadcasted_iota(jnp.int32, (RB5, VP), 1)
    bi_ref[0] = jnp.min(jnp.where(sc == tb, li, VP), axis=1, keepdims=True)


def kernel(hidden, W, b):
    f32 = jnp.float32
    lh = hidden[:, -1, :]
    Wp = jnp.pad(W.reshape(H, S, V),
                 ((0, 0), (0, 0), (0, VP - V))).reshape(H, S * VP)
    bp = jnp.pad(b.reshape(S, V), ((0, 0), (0, VP - V)),
                 constant_values=NEG).reshape(1, S * VP)

    l2, l4 = pl.pallas_call(
        _k1,
        grid=(S, NT),
        in_specs=[
            pl.BlockSpec((B, H), lambda s, t: (0, 0)),
            pl.BlockSpec((H, TP), lambda s, t: (0, s * NT + t)),
            pl.BlockSpec((1, TP), lambda s, t: (0, s * NT + t)),
        ],
        out_specs=[
            pl.BlockSpec((B, TP), lambda s, t: (0, s * NT + t)),
            pl.BlockSpec((B, NC, 128), lambda s, t: (s, t, 0)),
        ],
        out_shape=[
            jax.ShapeDtypeStruct((B, S * VP), f32),
            jax.ShapeDtypeStruct((S * B, NCH, 128), f32),
        ],
    )(lh, Wp, bp)

    gp = pl.pallas_call(
        _k3,
        grid=(S, B // RB),
        in_specs=[
            pl.BlockSpec((RB, NCH, 128),
                         lambda s, r: (s * (B // RB) + r, 0, 0)),
        ],
        out_specs=pl.BlockSpec((RB, NG, 128),
                               lambda s, r: (s * (B // RB) + r, 0, 0)),
        out_shape=jax.ShapeDtypeStruct((S * B, NG, 128), f32),
    )(l4)

    m_, th_, d_ = pl.pallas_call(
        _k4,
        grid=(S,),
        in_specs=[pl.BlockSpec((B, NG, 128), lambda s: (s, 0, 0))],
        out_specs=[
            pl.BlockSpec((1, B, 1), lambda s: (s, 0, 0)),
            pl.BlockSpec((1, B, 1), lambda s: (s, 0, 0)),
            pl.BlockSpec((1, B, 1), lambda s: (s, 0, 0)),
        ],
        out_shape=[
            jax.ShapeDtypeStruct((S, B, 1), f32),
            jax.ShapeDtypeStruct((S, B, 1), f32),
            jax.ShapeDtypeStruct((S, B, 1), f32),
        ],
    )(gp)

    u = jax.random.uniform(jax.random.key(42), (B, S, V), f32,
                           minval=1e-20, maxval=1.0)
    up = jnp.pad(u, ((0, 0), (0, 0), (0, VP - V)),
                 constant_values=0.5).reshape(B, S * VP)

    pp, bi = pl.pallas_call(
        _k5,
        grid=(S, B // RB5),
        in_specs=[
            pl.BlockSpec((RB5, VP), lambda s, r: (r, s)),
            pl.BlockSpec((RB5, VP), lambda s, r: (r, s)),
            pl.BlockSpec((1, RB5, 1), lambda s, r: (s, r, 0)),
            pl.BlockSpec((1, RB5, 1), lambda s, r: (s, r, 0)),
            pl.BlockSpec((1, RB5, 1), lambda s, r: (s, r, 0)),
        ],
        out_specs=[
            pl.BlockSpec((RB5, VP), lambda s, r: (r, s)),
            pl.BlockSpec((1, RB5, 1), lambda s, r: (s, r, 0)),
        ],
        out_shape=[
            jax.ShapeDtypeStruct((B, S * VP), f32),
            jax.ShapeDtypeStruct((S, B, 1), jnp.int32),
        ],
    )(l2, up, m_, th_, d_)

    probs = pp.reshape(B, S, VP)[:, :, :V]
    tokens = bi[:, :, 0].T
    return tokens, probs
